# Initial kernel scaffold; baseline (speedup 1.0000x reference)
import jax
import jax.numpy as jnp
from jax.experimental import pallas as pl


def kernel(x, edge_index, W0, b0, W1, b1, W2, b2):
    raise NotImplementedError("write your pallas kernel here")


# speedup vs baseline: 7.6015x; 7.6015x over previous
import functools

import jax
import jax.numpy as jnp
from jax import lax
from jax.experimental import pallas as pl
from jax.experimental.pallas import tpu as pltpu
from jax.experimental.pallas import tpu_sc as plsc

N = 10000
E = 320000
NPAD = 10240
EPAD = 327680
NBLK = EPAD // 128
NC = 2
NS = 16
ROWS_PER_TILE = NPAD // NS

_mesh = lambda: plsc.VectorSubcoreMesh(core_axis_name="c", subcore_axis_name="s")


def _sc_spmm(table, ridx_flat, cidx, zeros128, edge_split):
    nb_w = NBLK // (NC * NS) if edge_split else NBLK // NS

    @functools.partial(
        pl.kernel,
        out_type=jax.ShapeDtypeStruct((NC * NPAD, 128), jnp.float32),
        mesh=_mesh(),
        scratch_types=[
            pltpu.VMEM_SHARED((NPAD, 128), jnp.float32),
            pltpu.VMEM((128,), jnp.int32),
            pltpu.VMEM((128,), jnp.int32),
            pltpu.VMEM((128, 128), jnp.float32),
            pltpu.VMEM((32, 128), jnp.float32),
        ],
    )
    def k(table_hbm, ridx_hbm, cidx_hbm, zeros_hbm, out_hbm, agg_sh, rblk, cblk, gbuf, zb):
        c = lax.axis_index("c")
        s = lax.axis_index("s")
        if edge_split:
            off = (c * NS + s) * nb_w
            coff = off
        else:
            off = c * NBLK + s * nb_w
            coff = s * nb_w

        pltpu.sync_copy(zeros_hbm, zb)
        def zfire(i, carry):
            pltpu.sync_copy(zb, agg_sh.at[pl.ds(s * ROWS_PER_TILE + i * 32, 32)])
            return carry
        lax.fori_loop(0, ROWS_PER_TILE // 32, zfire, 0)
        plsc.subcore_barrier()

        def body(b, carry):
            pltpu.sync_copy(ridx_hbm.at[off + b], rblk)
            pltpu.sync_copy(cidx_hbm.at[coff + b], cblk)
            pltpu.sync_copy(table_hbm.at[rblk], gbuf)
            pltpu.sync_copy(gbuf, agg_sh.at[cblk], add=True)
            return carry
        lax.fori_loop(0, nb_w, body, 0)
        plsc.subcore_barrier()

        def wloop(i, carry):
            r = s * ROWS_PER_TILE + i * 128
            pltpu.sync_copy(agg_sh.at[pl.ds(r, 128)], gbuf)
            pltpu.sync_copy(gbuf, out_hbm.at[pl.ds(c * NPAD + r, 128)])
            return carry
        lax.fori_loop(0, ROWS_PER_TILE // 128, wloop, 0)

    return k(table, ridx_flat, cidx, zeros128)


_BN = 1024
_NB = NPAD // _BN


def _dinv_block(da, db):
    deg = da[:, 0:1] + db[:, 0:1]
    return jnp.where(deg > 0.0, lax.rsqrt(deg), 0.0)


def _tc_prep(degf, xp):
    def body(da_ref, db_ref, x_ref, o_ref):
        o_ref[...] = x_ref[...] * _dinv_block(da_ref[...], db_ref[...])

    return pl.pallas_call(
        body,
        grid=(_NB,),
        in_specs=[
            pl.BlockSpec((_BN, 128), lambda i: (i, 0)),
            pl.BlockSpec((_BN, 128), lambda i: (_NB + i, 0)),
            pl.BlockSpec((_BN, 128), lambda i: (i, 0)),
        ],
        out_specs=pl.BlockSpec((_BN, 128), lambda i: (i, 0)),
        out_shape=jax.ShapeDtypeStruct((NPAD, 128), jnp.float32),
    )(degf, degf, xp)


def _tc_layer0(degf, s0, W0, b0r):
    def body(da_ref, db_ref, sa_ref, sb_ref, w_ref, b_ref, o_ref):
        c = pl.program_id(1)
        dinv = _dinv_block(da_ref[...], db_ref[...])
        a = (sa_ref[...] + sb_ref[...]) * dinv
        h = lax.dot_general(a, w_ref[...], (((1,), (1,)), ((), ())),
                            preferred_element_type=jnp.float32)
        h = jnp.maximum(h + b_ref[pl.ds(c, 1), :], 0.0)
        o_ref[...] = h * dinv

    return pl.pallas_call(
        body,
        grid=(_NB, 2),
        in_specs=[
            pl.BlockSpec((_BN, 128), lambda i, c: (i, 0)),
            pl.BlockSpec((_BN, 128), lambda i, c: (_NB + i, 0)),
            pl.BlockSpec((_BN, 128), lambda i, c: (i, 0)),
            pl.BlockSpec((_BN, 128), lambda i, c: (_NB + i, 0)),
            pl.BlockSpec((128, 128), lambda i, c: (c, 0)),
            pl.BlockSpec((2, 128), lambda i, c: (0, 0)),
        ],
        out_specs=pl.BlockSpec((_BN, 128), lambda i, c: (c * _NB + i, 0)),
        out_shape=jax.ShapeDtypeStruct((NC * NPAD, 128), jnp.float32),
    )(degf, degf, s0, s0, W0, b0r)


def _tc_layer12(degf, s1, W1, b1r, W2):
    def body(da_ref, db_ref, sa_ref, sb_ref, w1_ref, b1_ref, w2_ref, o_ref):
        dinv = _dinv_block(da_ref[...], db_ref[...])
        a0 = sa_ref[...] * dinv
        a1 = sb_ref[...] * dinv
        t = lax.dot_general(a0, w1_ref[:, 0:128], (((1,), (1,)), ((), ())),
                            preferred_element_type=jnp.float32)
        t += lax.dot_general(a1, w1_ref[:, 128:256], (((1,), (1,)), ((), ())),
                             preferred_element_type=jnp.float32)
        h2 = jnp.maximum(b1_ref[...] - t, 0.0)
        g2 = lax.dot_general(h2, w2_ref[...], (((1,), (1,)), ((), ())),
                             preferred_element_type=jnp.float32)
        o_ref[...] = g2 * dinv

    return pl.pallas_call(
        body,
        grid=(_NB,),
        in_specs=[
            pl.BlockSpec((_BN, 128), lambda i: (i, 0)),
            pl.BlockSpec((_BN, 128), lambda i: (_NB + i, 0)),
            pl.BlockSpec((_BN, 128), lambda i: (i, 0)),
            pl.BlockSpec((_BN, 128), lambda i: (_NB + i, 0)),
            pl.BlockSpec((256, 256), lambda i: (0, 0)),
            pl.BlockSpec((1, 256), lambda i: (0, 0)),
            pl.BlockSpec((128, 256), lambda i: (0, 0)),
        ],
        out_specs=pl.BlockSpec((_BN, 128), lambda i: (i, 0)),
        out_shape=jax.ShapeDtypeStruct((NPAD, 128), jnp.float32),
    )(degf, degf, s1, s1, W1, b1r, W2)


def _tc_final(degf, s2, b2r):
    def body(da_ref, db_ref, sa_ref, sb_ref, b_ref, o_ref):
        dinv = _dinv_block(da_ref[...], db_ref[...])
        o_ref[...] = (sa_ref[...] + sb_ref[...]) * dinv + b_ref[...]

    return pl.pallas_call(
        body,
        grid=(_NB,),
        in_specs=[
            pl.BlockSpec((_BN, 128), lambda i: (i, 0)),
            pl.BlockSpec((_BN, 128), lambda i: (_NB + i, 0)),
            pl.BlockSpec((_BN, 128), lambda i: (i, 0)),
            pl.BlockSpec((_BN, 128), lambda i: (_NB + i, 0)),
            pl.BlockSpec((1, 128), lambda i: (0, 0)),
        ],
        out_specs=pl.BlockSpec((_BN, 128), lambda i: (i, 0)),
        out_shape=jax.ShapeDtypeStruct((NPAD, 128), jnp.float32),
    )(degf, degf, s2, s2, b2r)


def kernel(x, edge_index, W0, b0, W1, b1, W2, b2):
    row = edge_index[0].astype(jnp.int32)
    col = edge_index[1].astype(jnp.int32)
    pad = N + (jnp.arange(EPAD - E, dtype=jnp.int32) % (NPAD - N))
    ridx = jnp.concatenate([row, pad]).reshape(NBLK, 128)
    cidx = jnp.concatenate([col, pad]).reshape(NBLK, 128)
    ridx_sh = jnp.concatenate([ridx, ridx + NPAD], axis=0)
    xp = jnp.pad(x, ((0, NPAD - N), (0, 0)))
    zeros128 = jnp.zeros((32, 128), jnp.float32)
    ones_tab = jnp.ones((NPAD, 128), jnp.float32)
    b0r = b0.reshape(2, 128)
    b1r = b1.reshape(1, 256)
    b2r = b2.reshape(1, 128)

    degf = _sc_spmm(ones_tab, ridx, ridx, zeros128, True)
    g0 = _tc_prep(degf, xp)
    s0 = _sc_spmm(g0, ridx, cidx, zeros128, True)
    g1 = _tc_layer0(degf, s0, W0, b0r)
    s1 = _sc_spmm(g1, ridx_sh, cidx, zeros128, False)
    g2 = _tc_layer12(degf, s1, W1, b1r, W2)
    s2 = _sc_spmm(g2, ridx, cidx, zeros128, True)
    out = _tc_final(degf, s2, b2r)
    return out[:N]


# speedup vs baseline: 10.7423x; 1.4132x over previous
import functools

import jax
import jax.numpy as jnp
from jax import lax
from jax.experimental import pallas as pl
from jax.experimental.pallas import tpu as pltpu
from jax.experimental.pallas import tpu_sc as plsc

N = 10000
E = 320000
NPAD = 10240
EPAD = 327680
NBLK = EPAD // 128
NC = 2
NS = 16
ROWS_PER_TILE = NPAD // NS

_mesh = lambda: plsc.VectorSubcoreMesh(core_axis_name="c", subcore_axis_name="s")


def _sc_spmm(table, ridx_flat, cidx, zeros128, edge_split):
    nb_w = NBLK // (NC * NS) if edge_split else NBLK // NS

    @functools.partial(
        pl.kernel,
        out_type=jax.ShapeDtypeStruct((NC * NPAD, 128), jnp.float32),
        mesh=_mesh(),
        scratch_types=[
            pltpu.VMEM_SHARED((NPAD, 128), jnp.float32),
            pltpu.VMEM((128,), jnp.int32),
            pltpu.VMEM((128,), jnp.int32),
            pltpu.VMEM((128,), jnp.int32),
            pltpu.VMEM((128,), jnp.int32),
            pltpu.VMEM((128, 128), jnp.float32),
            pltpu.VMEM((128, 128), jnp.float32),
            pltpu.VMEM((32, 128), jnp.float32),
            pltpu.SemaphoreType.DMA,
            pltpu.SemaphoreType.DMA,
            pltpu.SemaphoreType.DMA,
            pltpu.SemaphoreType.DMA,
        ],
    )
    def k(table_hbm, ridx_hbm, cidx_hbm, zeros_hbm, out_hbm, agg_sh,
          rblk0, rblk1, cblk0, cblk1, gbuf0, gbuf1, zb,
          si0, si1, sg0, sg1):
        c = lax.axis_index("c")
        s = lax.axis_index("s")
        if edge_split:
            off = (c * NS + s) * nb_w
            coff = off
        else:
            off = c * NBLK + s * nb_w
            coff = s * nb_w

        pltpu.sync_copy(zeros_hbm, zb)
        def zfire(i, carry):
            pltpu.sync_copy(zb, agg_sh.at[pl.ds(s * ROWS_PER_TILE + i * 32, 32)])
            return carry
        lax.fori_loop(0, ROWS_PER_TILE // 32, zfire, 0)
        plsc.subcore_barrier()

        def body(g, carry):
            b = g * 2
            dr0 = pltpu.async_copy(ridx_hbm.at[off + b], rblk0, si0)
            dc0 = pltpu.async_copy(cidx_hbm.at[coff + b], cblk0, si0)
            dr1 = pltpu.async_copy(ridx_hbm.at[off + b + 1], rblk1, si1)
            dc1 = pltpu.async_copy(cidx_hbm.at[coff + b + 1], cblk1, si1)
            dr0.wait()
            dc0.wait()
            dg0 = pltpu.async_copy(table_hbm.at[rblk0], gbuf0, sg0)
            dr1.wait()
            dc1.wait()
            dg0.wait()
            dg1 = pltpu.async_copy(table_hbm.at[rblk1], gbuf1, sg1)
            pltpu.sync_copy(gbuf0, agg_sh.at[cblk0], add=True)
            dg1.wait()
            pltpu.sync_copy(gbuf1, agg_sh.at[cblk1], add=True)
            return carry
        lax.fori_loop(0, nb_w // 2, body, 0)
        plsc.subcore_barrier()

        def wloop(i, carry):
            r = s * ROWS_PER_TILE + i * 128
            pltpu.sync_copy(agg_sh.at[pl.ds(r, 128)], gbuf0)
            pltpu.sync_copy(gbuf0, out_hbm.at[pl.ds(c * NPAD + r, 128)])
            return carry
        lax.fori_loop(0, ROWS_PER_TILE // 128, wloop, 0)

    return k(table, ridx_flat, cidx, zeros128)


_BN = 1024
_NB = NPAD // _BN


def _dinv_block(da, db):
    deg = da[:, 0:1] + db[:, 0:1]
    return jnp.where(deg > 0.0, lax.rsqrt(deg), 0.0)


def _tc_prep(degf, xp):
    def body(da_ref, db_ref, x_ref, o_ref):
        o_ref[...] = x_ref[...] * _dinv_block(da_ref[...], db_ref[...])

    return pl.pallas_call(
        body,
        grid=(_NB,),
        in_specs=[
            pl.BlockSpec((_BN, 128), lambda i: (i, 0)),
            pl.BlockSpec((_BN, 128), lambda i: (_NB + i, 0)),
            pl.BlockSpec((_BN, 128), lambda i: (i, 0)),
        ],
        out_specs=pl.BlockSpec((_BN, 128), lambda i: (i, 0)),
        out_shape=jax.ShapeDtypeStruct((NPAD, 128), jnp.float32),
    )(degf, degf, xp)


def _tc_layer0(degf, s0, W0, b0r):
    def body(da_ref, db_ref, sa_ref, sb_ref, w_ref, b_ref, o_ref):
        c = pl.program_id(1)
        dinv = _dinv_block(da_ref[...], db_ref[...])
        a = (sa_ref[...] + sb_ref[...]) * dinv
        h = lax.dot_general(a, w_ref[...], (((1,), (1,)), ((), ())),
                            preferred_element_type=jnp.float32)
        h = jnp.maximum(h + b_ref[pl.ds(c, 1), :], 0.0)
        o_ref[...] = h * dinv

    return pl.pallas_call(
        body,
        grid=(_NB, 2),
        in_specs=[
            pl.BlockSpec((_BN, 128), lambda i, c: (i, 0)),
            pl.BlockSpec((_BN, 128), lambda i, c: (_NB + i, 0)),
            pl.BlockSpec((_BN, 128), lambda i, c: (i, 0)),
            pl.BlockSpec((_BN, 128), lambda i, c: (_NB + i, 0)),
            pl.BlockSpec((128, 128), lambda i, c: (c, 0)),
            pl.BlockSpec((2, 128), lambda i, c: (0, 0)),
        ],
        out_specs=pl.BlockSpec((_BN, 128), lambda i, c: (c * _NB + i, 0)),
        out_shape=jax.ShapeDtypeStruct((NC * NPAD, 128), jnp.float32),
    )(degf, degf, s0, s0, W0, b0r)


def _tc_layer12(degf, s1, W1, b1r, W2):
    def body(da_ref, db_ref, sa_ref, sb_ref, w1_ref, b1_ref, w2_ref, o_ref):
        dinv = _dinv_block(da_ref[...], db_ref[...])
        a0 = sa_ref[...] * dinv
        a1 = sb_ref[...] * dinv
        t = lax.dot_general(a0, w1_ref[:, 0:128], (((1,), (1,)), ((), ())),
                            preferred_element_type=jnp.float32)
        t += lax.dot_general(a1, w1_ref[:, 128:256], (((1,), (1,)), ((), ())),
                             preferred_element_type=jnp.float32)
        h2 = jnp.maximum(b1_ref[...] - t, 0.0)
        g2 = lax.dot_general(h2, w2_ref[...], (((1,), (1,)), ((), ())),
                             preferred_element_type=jnp.float32)
        o_ref[...] = g2 * dinv

    return pl.pallas_call(
        body,
        grid=(_NB,),
        in_specs=[
            pl.BlockSpec((_BN, 128), lambda i: (i, 0)),
            pl.BlockSpec((_BN, 128), lambda i: (_NB + i, 0)),
            pl.BlockSpec((_BN, 128), lambda i: (i, 0)),
            pl.BlockSpec((_BN, 128), lambda i: (_NB + i, 0)),
            pl.BlockSpec((256, 256), lambda i: (0, 0)),
            pl.BlockSpec((1, 256), lambda i: (0, 0)),
            pl.BlockSpec((128, 256), lambda i: (0, 0)),
        ],
        out_specs=pl.BlockSpec((_BN, 128), lambda i: (i, 0)),
        out_shape=jax.ShapeDtypeStruct((NPAD, 128), jnp.float32),
    )(degf, degf, s1, s1, W1, b1r, W2)


def _tc_final(degf, s2, b2r):
    def body(da_ref, db_ref, sa_ref, sb_ref, b_ref, o_ref):
        dinv = _dinv_block(da_ref[...], db_ref[...])
        o_ref[...] = (sa_ref[...] + sb_ref[...]) * dinv + b_ref[...]

    return pl.pallas_call(
        body,
        grid=(_NB,),
        in_specs=[
            pl.BlockSpec((_BN, 128), lambda i: (i, 0)),
            pl.BlockSpec((_BN, 128), lambda i: (_NB + i, 0)),
            pl.BlockSpec((_BN, 128), lambda i: (i, 0)),
            pl.BlockSpec((_BN, 128), lambda i: (_NB + i, 0)),
            pl.BlockSpec((1, 128), lambda i: (0, 0)),
        ],
        out_specs=pl.BlockSpec((_BN, 128), lambda i: (i, 0)),
        out_shape=jax.ShapeDtypeStruct((NPAD, 128), jnp.float32),
    )(degf, degf, s2, s2, b2r)


def kernel(x, edge_index, W0, b0, W1, b1, W2, b2):
    row = edge_index[0].astype(jnp.int32)
    col = edge_index[1].astype(jnp.int32)
    pad = N + (jnp.arange(EPAD - E, dtype=jnp.int32) % (NPAD - N))
    ridx = jnp.concatenate([row, pad]).reshape(NBLK, 128)
    cidx = jnp.concatenate([col, pad]).reshape(NBLK, 128)
    ridx_sh = jnp.concatenate([ridx, ridx + NPAD], axis=0)
    xp = jnp.pad(x, ((0, NPAD - N), (0, 0)))
    zeros128 = jnp.zeros((32, 128), jnp.float32)
    ones_tab = jnp.ones((NPAD, 128), jnp.float32)
    b0r = b0.reshape(2, 128)
    b1r = b1.reshape(1, 256)
    b2r = b2.reshape(1, 128)

    degf = _sc_spmm(ones_tab, ridx, ridx, zeros128, True)
    g0 = _tc_prep(degf, xp)
    s0 = _sc_spmm(g0, ridx, cidx, zeros128, True)
    g1 = _tc_layer0(degf, s0, W0, b0r)
    s1 = _sc_spmm(g1, ridx_sh, cidx, zeros128, False)
    g2 = _tc_layer12(degf, s1, W1, b1r, W2)
    s2 = _sc_spmm(g2, ridx, cidx, zeros128, True)
    out = _tc_final(degf, s2, b2r)
    return out[:N]
